# baseline (speedup 1.0000x reference)
import functools
import math

import jax
import jax.numpy as jnp
from jax.experimental import pallas as pl
from jax.experimental.pallas import tpu as pltpu

BLK = 1024

_TWO_PI_HI = 6.28125
_TWO_PI_LO = 2.0 * math.pi - 6.28125
_INV_2PI = 1.0 / (2.0 * math.pi)
_MAGIC = 1.5 * 2.0**23
_S1 = 0.9999793367663286
_S3 = -0.16662434262541412
_S5 = 0.00830897441021473
_S7 = -0.00019264897422000687
_S9 = 2.1478432028210204e-06


def _fast_sin(t):
    k = (t * _INV_2PI + _MAGIC) - _MAGIC
    r = (t - k * _TWO_PI_HI) - k * _TWO_PI_LO
    r2 = r * r
    p = _S7 + r2 * _S9
    p = _S5 + r2 * p
    p = _S3 + r2 * p
    p = _S1 + r2 * p
    return r * p


def _tc_body(p_ref, w_ref, x_ref, o_ref):
    t = jnp.dot(
        p_ref[...],
        w_ref[...],
        preferred_element_type=jnp.float32,
        precision=jax.lax.Precision.HIGHEST,
    )
    o_ref[...] = x_ref[...] + _fast_sin(t)


@functools.lru_cache(maxsize=None)
def _build_tc(n_rows, d_model):
    grid = (n_rows // BLK,)
    return pl.pallas_call(
        _tc_body,
        grid=grid,
        in_specs=[
            pl.BlockSpec((BLK, 8), lambda i: (i, 0)),
            pl.BlockSpec((8, d_model), lambda i: (0, 0)),
            pl.BlockSpec((BLK, d_model), lambda i: (i, 0)),
        ],
        out_specs=pl.BlockSpec((BLK, d_model), lambda i: (i, 0)),
        out_shape=jax.ShapeDtypeStruct((n_rows, d_model), jnp.float32),
    )


def kernel(x, position, pe):
    b, s, d = x.shape
    n = b * s
    half = d // 2
    div_term = jnp.exp(
        jnp.arange(0, d, 2, dtype=jnp.float32) * (-math.log(10000.0) / d)
    )
    w = jnp.repeat(div_term, 2).reshape(1, d)
    ph = jnp.tile(jnp.asarray([0.0, math.pi / 2], jnp.float32), half).reshape(1, d)
    w_aug = jnp.concatenate([w, ph, jnp.zeros((6, d), jnp.float32)], axis=0)
    p_f = position.reshape(n, 1).astype(jnp.float32)
    p_aug = jnp.concatenate(
        [p_f, jnp.ones((n, 1), jnp.float32), jnp.zeros((n, 6), jnp.float32)], axis=1
    )
    out = _build_tc(n, d)(p_aug, w_aug, x.reshape(n, d))
    return out.reshape(b, s, d)


# speedup vs baseline: 1.3622x; 1.3622x over previous
import functools
import math

import jax
import jax.numpy as jnp
from jax.experimental import pallas as pl
from jax.experimental.pallas import tpu as pltpu

BLK = 1024

_TWO_PI_HI = 6.28125
_TWO_PI_LO = 2.0 * math.pi - 6.28125
_INV_2PI = 1.0 / (2.0 * math.pi)
_MAGIC = 1.5 * 2.0**23
_S1 = 0.9999793367663286
_S3 = -0.16662434262541412
_S5 = 0.00830897441021473
_S7 = -0.00019264897422000687
_S9 = 2.1478432028210204e-06


def _fast_sin(t):
    k = (t * _INV_2PI + _MAGIC) - _MAGIC
    r = (t - k * _TWO_PI_HI) - k * _TWO_PI_LO
    r2 = r * r
    p = _S7 + r2 * _S9
    p = _S5 + r2 * p
    p = _S3 + r2 * p
    p = _S1 + r2 * p
    return r * p


def _tc_body(p_ref, w_ref, x_ref, o_ref):
    t = jnp.dot(p_ref[...], w_ref[...], preferred_element_type=jnp.float32)
    o_ref[...] = x_ref[...] + _fast_sin(t)


@functools.lru_cache(maxsize=None)
def _build_tc(n_rows, d_model):
    grid = (n_rows // BLK,)
    return pl.pallas_call(
        _tc_body,
        grid=grid,
        in_specs=[
            pl.BlockSpec((BLK, 16), lambda i: (i, 0)),
            pl.BlockSpec((16, d_model), lambda i: (0, 0)),
            pl.BlockSpec((BLK, d_model), lambda i: (i, 0)),
        ],
        out_specs=pl.BlockSpec((BLK, d_model), lambda i: (i, 0)),
        out_shape=jax.ShapeDtypeStruct((n_rows, d_model), jnp.float32),
    )


def kernel(x, position, pe):
    b, s, d = x.shape
    n = b * s
    half = d // 2
    f32, bf16 = jnp.float32, jnp.bfloat16
    div_term = jnp.exp(jnp.arange(0, d, 2, dtype=f32) * (-math.log(10000.0) / d))
    w = jnp.repeat(div_term, 2)
    ph = jnp.tile(jnp.asarray([0.0, math.pi / 2], f32), half)
    w1 = w.astype(bf16)
    w2 = (w - w1.astype(f32)).astype(bf16)
    w3 = (w - w1.astype(f32) - w2.astype(f32)).astype(bf16)
    ph1 = ph.astype(bf16)
    ph2 = (ph - ph1.astype(f32)).astype(bf16)
    zed = jnp.zeros((d,), bf16)
    w_aug = jnp.stack(
        [
            (w1.astype(f32) * 64).astype(bf16),
            (w2.astype(f32) * 64).astype(bf16),
            (w3.astype(f32) * 64).astype(bf16),
            w1, w2, w3, ph1, ph2,
            zed, zed, zed, zed, zed, zed, zed, zed,
        ],
        axis=0,
    )
    p_i = position.reshape(n)
    p_hi = (p_i // 64).astype(f32).astype(bf16)
    p_lo = (p_i % 64).astype(f32).astype(bf16)
    one = jnp.ones((n,), bf16)
    zn = jnp.zeros((n,), bf16)
    p_aug = jnp.stack(
        [p_hi, p_hi, p_hi, p_lo, p_lo, p_lo, one, one,
         zn, zn, zn, zn, zn, zn, zn, zn],
        axis=1,
    )
    out = _build_tc(n, d)(p_aug, w_aug, x.reshape(n, d))
    return out.reshape(b, s, d)
